# Initial kernel scaffold; baseline (speedup 1.0000x reference)
import jax
import jax.numpy as jnp
from jax.experimental import pallas as pl


def kernel(feats_atom, feats_bond, feats_global, batch_atom, batch_bond, W_atom, b_atom, W_bond, b_bond):
    raise NotImplementedError("write your pallas kernel here")


# speedup vs baseline: 5.7861x; 5.7861x over previous
import jax
import jax.numpy as jnp
from jax.experimental import pallas as pl

_N, _D, _B, _DG = 100000, 128, 512, 64
_R = 1000
_G = _N // _R


def _body(fa, ba, fb, bb, wa, wb, bia, bib, gl, out_ref):
    i = pl.program_id(0)

    @pl.when(i == 0)
    def _init():
        out_ref[...] = jnp.zeros_like(out_ref)
        out_ref[:, 2 * _D:] = gl[...]

    for (f_ref, b_ref, w_ref, bias_ref, col) in (
        (fa, ba, wa, bia, 0),
        (fb, bb, wb, bib, _D),
    ):
        f = f_ref[...]
        wrow = w_ref[...]
        bias = bias_ref[0, 0]
        logit = jnp.sum(f * wrow, axis=1, keepdims=True) + bias
        wgt = 1.0 / (1.0 + jnp.exp(-logit))
        weighted = f * wgt
        ids = b_ref[0, 0, :]
        seg = jax.lax.broadcasted_iota(jnp.int32, (_B, _R), 0)
        oh = (seg == ids[None, :]).astype(jnp.float32)
        contrib = jax.lax.dot(oh, weighted,
                              preferred_element_type=jnp.float32)
        out_ref[:, col:col + _D] = out_ref[:, col:col + _D] + contrib


def kernel(feats_atom, feats_bond, feats_global, batch_atom, batch_bond,
           W_atom, b_atom, W_bond, b_bond):
    ba = batch_atom.astype(jnp.int32).reshape(_G, 1, _R)
    bb = batch_bond.astype(jnp.int32).reshape(_G, 1, _R)
    wa = W_atom.reshape(1, _D)
    wb = W_bond.reshape(1, _D)
    bia = b_atom.reshape(1, 1)
    bib = b_bond.reshape(1, 1)
    return pl.pallas_call(
        _body,
        grid=(_G,),
        in_specs=[
            pl.BlockSpec((_R, _D), lambda i: (i, 0)),
            pl.BlockSpec((1, 1, _R), lambda i: (i, 0, 0)),
            pl.BlockSpec((_R, _D), lambda i: (i, 0)),
            pl.BlockSpec((1, 1, _R), lambda i: (i, 0, 0)),
            pl.BlockSpec((1, _D), lambda i: (0, 0)),
            pl.BlockSpec((1, _D), lambda i: (0, 0)),
            pl.BlockSpec((1, 1), lambda i: (0, 0)),
            pl.BlockSpec((1, 1), lambda i: (0, 0)),
            pl.BlockSpec((_B, _DG), lambda i: (0, 0)),
        ],
        out_specs=pl.BlockSpec((_B, 2 * _D + _DG), lambda i: (0, 0)),
        out_shape=jax.ShapeDtypeStruct((_B, 2 * _D + _DG), jnp.float32),
    )(feats_atom, ba, feats_bond, bb, wa, wb, bia, bib, feats_global)
